# baseline (speedup 1.0000x reference)
import jax
import jax.numpy as jnp
from jax.experimental import pallas as pl

_B, _A, _H, _W = 64, 8, 64, 64
_N = _B * _A
_D = 512
_K = 3

_C1_BLK = 8
_C2_BLK = 16


def _conv1_body(x5_ref, w_ref, b_ref, o_ref):
    x5 = x5_ref[...].astype(jnp.bfloat16).astype(jnp.float32)
    wf = w_ref[...].astype(jnp.bfloat16).astype(jnp.float32)
    acc = jnp.zeros((_C1_BLK, 16, 16, 64), jnp.float32)
    for kh in range(7):
        ah, rh = divmod(kh, 4)
        for kw in range(7):
            aw, rw = divmod(kw, 4)
            tap = x5[:, ah:ah + 16, rh, aw:aw + 16, rw]
            wv = wf[kh * 7 + kw:kh * 7 + kw + 1, :]
            acc = acc + tap[:, :, :, None] * wv
    o_ref[...] = jnp.maximum(acc + b_ref[...], 0.0)


def _conv2_body(x6_ref, w_ref, b_ref, o_ref):
    x6 = x6_ref[...]
    acc = jnp.zeros((_C2_BLK * 16, 512), jnp.float32)
    for kh in range(3):
        for kw in range(3):
            tap = x6[:, :, kh, :, kw, :].reshape(_C2_BLK * 16, 64)
            acc = acc + jnp.dot(tap.astype(jnp.bfloat16),
                                w_ref[kh * 3 + kw].astype(jnp.bfloat16),
                                preferred_element_type=jnp.float32)
    acc = jnp.maximum(acc + b_ref[...], 0.0)
    pooled = jnp.mean(acc.reshape(_C2_BLK, 16, 512), axis=1)
    o_ref[...] = pooled


def _adj_body(enc_ref, p_ref):
    jio = jax.lax.broadcasted_iota(jnp.int32, (_B, _A), 1)
    def unit_row(j):
        rj = enc_ref[:, j:j + 1, :]
        ssj = jnp.sum(rj * rj, axis=-1)
        nj = rj / (jnp.sqrt(ssj)[:, :, None] + 1e-12)
        return nj.astype(jnp.bfloat16).astype(jnp.float32)

    for i in range(_A):
        ni = unit_row(i)
        cols = []
        for j in range(_A):
            cols.append(jnp.sum(ni * unit_row(j), axis=-1))
        sims_i = jnp.concatenate(cols, axis=1)
        rank_i = jnp.zeros((_B, _A), jnp.float32)
        for k in range(_A):
            s_k = sims_i[:, k:k + 1]
            gt = (s_k > sims_i).astype(jnp.float32)
            tie = ((s_k == sims_i) & (jio > k)).astype(jnp.float32)
            rank_i = rank_i + gt + tie
        p_i = jnp.where(rank_i < float(_K), 1.0 / _K, 0.0)
        p_ref[:, i:i + 1, :] = p_i[:, None, :]


def _sage_body(h_ref, p_ref, wl_ref, wr_ref, b_ref, o_ref, *, act, head):
    h = h_ref[...]
    h3 = h.reshape(_B, _A, _D)
    rows = []
    for i in range(_A):
        m_i = jnp.zeros((_B, 1, _D), jnp.float32)
        for j in range(_A):
            pij = p_ref[:, i:i + 1, j:j + 1]
            m_i = m_i + pij * h3[:, j:j + 1, :]
        rows.append(m_i)
    m3 = jnp.concatenate(rows, axis=1)
    mean = m3.reshape(_N, _D)
    hn = (jnp.dot(mean.astype(jnp.bfloat16),
                  wl_ref[...].astype(jnp.bfloat16),
                  preferred_element_type=jnp.float32)
          + jnp.dot(h.astype(jnp.bfloat16),
                    wr_ref[...].astype(jnp.bfloat16),
                    preferred_element_type=jnp.float32)
          + b_ref[...])
    if act:
        hn = jnp.where(hn >= 0.0, hn, 0.2 * hn)
    if head is None:
        o_ref[...] = hn
    else:
        cw, cb = head
        pooled = jnp.mean(hn.reshape(_B, _A, _D), axis=1)
        o_ref[...] = (jnp.dot(pooled.astype(jnp.bfloat16),
                              cw[...].astype(jnp.bfloat16),
                              preferred_element_type=jnp.float32) + cb[...])


def _sage_head_body(h_ref, p_ref, wl_ref, wr_ref, b_ref, cw_ref, cb_ref, o_ref):
    _sage_body(h_ref, p_ref, wl_ref, wr_ref, b_ref, o_ref,
               act=False, head=(cw_ref, cb_ref))


def kernel(x, conv1_w, conv1_b, conv2_w, conv2_b,
           sage1_wl, sage1_wr, sage1_b,
           sage2_wl, sage2_wr, sage2_b,
           sage3_wl, sage3_wr, sage3_b,
           cls_w, cls_b):
    xi = x.reshape(_N, _H, _W)
    xp = jnp.pad(xi, ((0, 0), (1, 3), (1, 3)))
    x5 = xp.reshape(_N, 17, 4, 17, 4)
    w1 = conv1_w.reshape(64, 49).T
    b1 = conv1_b.reshape(1, 64)

    y1 = pl.pallas_call(
        _conv1_body,
        grid=(_N // _C1_BLK,),
        in_specs=[
            pl.BlockSpec((_C1_BLK, 17, 4, 17, 4), lambda i: (i, 0, 0, 0, 0)),
            pl.BlockSpec((49, 64), lambda i: (0, 0)),
            pl.BlockSpec((1, 64), lambda i: (0, 0)),
        ],
        out_specs=pl.BlockSpec((_C1_BLK, 16, 16, 64), lambda i: (i, 0, 0, 0)),
        out_shape=jax.ShapeDtypeStruct((_N, 16, 16, 64), jnp.float32),
    )(x5, w1, b1)

    x6 = y1.reshape(_N, 4, 4, 4, 4, 64)
    w2 = conv2_w.transpose(2, 3, 1, 0).reshape(9, 64, 512)
    b2 = conv2_b.reshape(1, 512)

    enc = pl.pallas_call(
        _conv2_body,
        grid=(_N // _C2_BLK,),
        in_specs=[
            pl.BlockSpec((_C2_BLK, 4, 4, 4, 4, 64),
                         lambda i: (i, 0, 0, 0, 0, 0)),
            pl.BlockSpec((9, 64, 512), lambda i: (0, 0, 0)),
            pl.BlockSpec((1, 512), lambda i: (0, 0)),
        ],
        out_specs=pl.BlockSpec((_C2_BLK, 512), lambda i: (i, 0)),
        out_shape=jax.ShapeDtypeStruct((_N, _D), jnp.float32),
    )(x6, w2, b2)

    p = pl.pallas_call(
        _adj_body,
        out_shape=jax.ShapeDtypeStruct((_B, _A, _A), jnp.float32),
    )(enc.reshape(_B, _A, _D))

    import functools as _ft
    h = enc
    for wl, wr, b in ((sage1_wl, sage1_wr, sage1_b),
                      (sage2_wl, sage2_wr, sage2_b)):
        h = pl.pallas_call(
            _ft.partial(_sage_body, act=True, head=None),
            out_shape=jax.ShapeDtypeStruct((_N, _D), jnp.float32),
        )(h, p, wl.T, wr.T, b.reshape(1, _D))

    out = pl.pallas_call(
        _sage_head_body,
        out_shape=jax.ShapeDtypeStruct((_B, 2), jnp.float32),
    )(h, p, sage3_wl.T, sage3_wr.T, sage3_b.reshape(1, _D),
      cls_w.T, cls_b.reshape(1, 2))
    return out


# speedup vs baseline: 2.5792x; 2.5792x over previous
import jax
import jax.numpy as jnp
from jax.experimental import pallas as pl

_B, _A, _H, _W = 64, 8, 64, 64
_N = _B * _A
_D = 512
_K = 3

_C1_BLK = 16
_C2_BLK = 16


def _conv1_body(x4_ref, g_ref, b_ref, o_ref):
    x4 = x4_ref[...]
    acc = jnp.zeros((_C1_BLK * 16, 1024), jnp.float32)
    for kh in range(7):
        ah, rh = divmod(kh, 4)
        xk = x4[:, ah:ah + 16, rh, :].reshape(_C1_BLK * 16, 68)
        acc = acc + jnp.dot(xk.astype(jnp.bfloat16), g_ref[kh],
                            preferred_element_type=jnp.float32)
    o_ref[...] = jnp.maximum(acc + b_ref[...], 0.0)


def _conv2_body(x6_ref, w_ref, b_ref, o_ref):
    x6 = x6_ref[...]
    acc = jnp.zeros((_C2_BLK * 16, 512), jnp.float32)
    for kh in range(3):
        for kw in range(3):
            tap = x6[:, :, kh, :, kw, :].reshape(_C2_BLK * 16, 64)
            acc = acc + jnp.dot(tap.astype(jnp.bfloat16),
                                w_ref[kh * 3 + kw].astype(jnp.bfloat16),
                                preferred_element_type=jnp.float32)
    acc = jnp.maximum(acc + b_ref[...], 0.0)
    pooled = jnp.mean(acc.reshape(_C2_BLK, 16, 512), axis=1)
    o_ref[...] = pooled


def _adj_body(enc_ref, p_ref):
    jio = jax.lax.broadcasted_iota(jnp.int32, (_B, _A), 1)
    def unit_row(j):
        rj = enc_ref[:, j:j + 1, :]
        ssj = jnp.sum(rj * rj, axis=-1)
        nj = rj / (jnp.sqrt(ssj)[:, :, None] + 1e-12)
        return nj.astype(jnp.bfloat16).astype(jnp.float32)

    for i in range(_A):
        ni = unit_row(i)
        cols = []
        for j in range(_A):
            cols.append(jnp.sum(ni * unit_row(j), axis=-1))
        sims_i = jnp.concatenate(cols, axis=1)
        rank_i = jnp.zeros((_B, _A), jnp.float32)
        for k in range(_A):
            s_k = sims_i[:, k:k + 1]
            gt = (s_k > sims_i).astype(jnp.float32)
            tie = ((s_k == sims_i) & (jio > k)).astype(jnp.float32)
            rank_i = rank_i + gt + tie
        p_i = jnp.where(rank_i < float(_K), 1.0 / _K, 0.0)
        p_ref[:, i:i + 1, :] = p_i[:, None, :]


def _sage_body(h_ref, p_ref, wl_ref, wr_ref, b_ref, o_ref, *, act, head):
    h = h_ref[...]
    h3 = h.reshape(_B, _A, _D)
    rows = []
    for i in range(_A):
        m_i = jnp.zeros((_B, 1, _D), jnp.float32)
        for j in range(_A):
            pij = p_ref[:, i:i + 1, j:j + 1]
            m_i = m_i + pij * h3[:, j:j + 1, :]
        rows.append(m_i)
    m3 = jnp.concatenate(rows, axis=1)
    mean = m3.reshape(_N, _D)
    hn = (jnp.dot(mean.astype(jnp.bfloat16),
                  wl_ref[...].astype(jnp.bfloat16),
                  preferred_element_type=jnp.float32)
          + jnp.dot(h.astype(jnp.bfloat16),
                    wr_ref[...].astype(jnp.bfloat16),
                    preferred_element_type=jnp.float32)
          + b_ref[...])
    if act:
        hn = jnp.where(hn >= 0.0, hn, 0.2 * hn)
    if head is None:
        o_ref[...] = hn
    else:
        cw, cb = head
        pooled = jnp.mean(hn.reshape(_B, _A, _D), axis=1)
        o_ref[...] = (jnp.dot(pooled.astype(jnp.bfloat16),
                              cw[...].astype(jnp.bfloat16),
                              preferred_element_type=jnp.float32) + cb[...])


def _sage_head_body(h_ref, p_ref, wl_ref, wr_ref, b_ref, cw_ref, cb_ref, o_ref):
    _sage_body(h_ref, p_ref, wl_ref, wr_ref, b_ref, o_ref,
               act=False, head=(cw_ref, cb_ref))


def kernel(x, conv1_w, conv1_b, conv2_w, conv2_b,
           sage1_wl, sage1_wr, sage1_b,
           sage2_wl, sage2_wr, sage2_b,
           sage3_wl, sage3_wr, sage3_b,
           cls_w, cls_b):
    xi = x.reshape(_N, _H, _W)
    xp = jnp.pad(xi, ((0, 0), (1, 3), (1, 3)))
    x4 = xp.reshape(_N, 17, 4, 68)
    kh_i = jnp.arange(7)[:, None, None]
    kw_i = jnp.arange(7)[None, :, None]
    ox_i = jnp.arange(16)[None, None, :]
    w1v = jnp.broadcast_to(
        conv1_w.reshape(64, 7, 7).transpose(1, 2, 0)[:, :, None, :],
        (7, 7, 16, 64))
    g = jnp.zeros((7, 68, 16, 64), jnp.float32)
    g = g.at[jnp.broadcast_to(kh_i, (7, 7, 16)),
             jnp.broadcast_to(4 * ox_i + kw_i, (7, 7, 16)),
             jnp.broadcast_to(ox_i, (7, 7, 16)), :].set(w1v)
    g = g.reshape(7, 68, 1024).astype(jnp.bfloat16)
    b1 = jnp.tile(conv1_b, 16).reshape(1, 1024)

    y1 = pl.pallas_call(
        _conv1_body,
        grid=(_N // _C1_BLK,),
        in_specs=[
            pl.BlockSpec((_C1_BLK, 17, 4, 68), lambda i: (i, 0, 0, 0)),
            pl.BlockSpec((7, 68, 1024), lambda i: (0, 0, 0)),
            pl.BlockSpec((1, 1024), lambda i: (0, 0)),
        ],
        out_specs=pl.BlockSpec((_C1_BLK * 16, 1024), lambda i: (i, 0)),
        out_shape=jax.ShapeDtypeStruct((_N * 16, 1024), jnp.float32),
    )(x4, g, b1)

    x6 = y1.reshape(_N, 4, 4, 4, 4, 64)
    w2 = conv2_w.transpose(2, 3, 1, 0).reshape(9, 64, 512)
    b2 = conv2_b.reshape(1, 512)

    enc = pl.pallas_call(
        _conv2_body,
        grid=(_N // _C2_BLK,),
        in_specs=[
            pl.BlockSpec((_C2_BLK, 4, 4, 4, 4, 64),
                         lambda i: (i, 0, 0, 0, 0, 0)),
            pl.BlockSpec((9, 64, 512), lambda i: (0, 0, 0)),
            pl.BlockSpec((1, 512), lambda i: (0, 0)),
        ],
        out_specs=pl.BlockSpec((_C2_BLK, 512), lambda i: (i, 0)),
        out_shape=jax.ShapeDtypeStruct((_N, _D), jnp.float32),
    )(x6, w2, b2)

    p = pl.pallas_call(
        _adj_body,
        out_shape=jax.ShapeDtypeStruct((_B, _A, _A), jnp.float32),
    )(enc.reshape(_B, _A, _D))

    import functools as _ft
    h = enc
    for wl, wr, b in ((sage1_wl, sage1_wr, sage1_b),
                      (sage2_wl, sage2_wr, sage2_b)):
        h = pl.pallas_call(
            _ft.partial(_sage_body, act=True, head=None),
            out_shape=jax.ShapeDtypeStruct((_N, _D), jnp.float32),
        )(h, p, wl.T, wr.T, b.reshape(1, _D))

    out = pl.pallas_call(
        _sage_head_body,
        out_shape=jax.ShapeDtypeStruct((_B, 2), jnp.float32),
    )(h, p, sage3_wl.T, sage3_wr.T, sage3_b.reshape(1, _D),
      cls_w.T, cls_b.reshape(1, 2))
    return out


# speedup vs baseline: 2.7429x; 1.0635x over previous
import jax
import jax.numpy as jnp
from jax.experimental import pallas as pl

_B, _A, _H, _W = 64, 8, 64, 64
_N = _B * _A
_D = 512
_K = 3

_C1_BLK = 16
_C2_BLK = 16


def _conv1_body(x4_ref, g_ref, b_ref, o_ref):
    x4 = x4_ref[...]
    acc = jnp.zeros((_C1_BLK * 16, 1024), jnp.float32)
    for kh in range(7):
        ah, rh = divmod(kh, 4)
        xk = x4[:, ah:ah + 16, rh, :].reshape(_C1_BLK * 16, 68)
        acc = acc + jnp.dot(xk.astype(jnp.bfloat16), g_ref[kh],
                            preferred_element_type=jnp.float32)
    o_ref[...] = jnp.maximum(acc + b_ref[...], 0.0)


def _conv2_body(x6_ref, w_ref, b_ref, o_ref):
    x6 = x6_ref[...]
    acc = jnp.zeros((_C2_BLK * 16, 512), jnp.float32)
    for kh in range(3):
        for kw in range(3):
            tap = x6[:, :, kh, :, kw, :].reshape(_C2_BLK * 16, 64)
            acc = acc + jnp.dot(tap.astype(jnp.bfloat16),
                                w_ref[kh * 3 + kw].astype(jnp.bfloat16),
                                preferred_element_type=jnp.float32)
    acc = jnp.maximum(acc + b_ref[...], 0.0)
    pooled = jnp.mean(acc.reshape(_C2_BLK, 16, 512), axis=1)
    o_ref[...] = pooled


def _adj_body(enc_ref, p_ref):
    jio = jax.lax.broadcasted_iota(jnp.int32, (_B, _A), 1)
    def unit_row(j):
        rj = enc_ref[:, j:j + 1, :]
        ssj = jnp.sum(rj * rj, axis=-1)
        nj = rj / (jnp.sqrt(ssj)[:, :, None] + 1e-12)
        return nj.astype(jnp.bfloat16).astype(jnp.float32)

    units = [unit_row(j) for j in range(_A)]
    for i in range(_A):
        ni = units[i]
        cols = []
        for j in range(_A):
            cols.append(jnp.sum(ni * units[j], axis=-1))
        sims_i = jnp.concatenate(cols, axis=1)
        rank_i = jnp.zeros((_B, _A), jnp.float32)
        for k in range(_A):
            s_k = sims_i[:, k:k + 1]
            gt = (s_k > sims_i).astype(jnp.float32)
            tie = ((s_k == sims_i) & (jio > k)).astype(jnp.float32)
            rank_i = rank_i + gt + tie
        p_i = jnp.where(rank_i < float(_K), 1.0 / _K, 0.0)
        p_ref[:, i:i + 1, :] = p_i[:, None, :]


def _sage_body(h_ref, p_ref, wl_ref, wr_ref, b_ref, o_ref, *, act, head):
    h = h_ref[...]
    h3 = h.reshape(_B, _A, _D)
    rows = []
    for i in range(_A):
        m_i = jnp.zeros((_B, 1, _D), jnp.float32)
        for j in range(_A):
            pij = p_ref[:, i:i + 1, j:j + 1]
            m_i = m_i + pij * h3[:, j:j + 1, :]
        rows.append(m_i)
    m3 = jnp.concatenate(rows, axis=1)
    mean = m3.reshape(_N, _D)
    hn = (jnp.dot(mean.astype(jnp.bfloat16),
                  wl_ref[...].astype(jnp.bfloat16),
                  preferred_element_type=jnp.float32)
          + jnp.dot(h.astype(jnp.bfloat16),
                    wr_ref[...].astype(jnp.bfloat16),
                    preferred_element_type=jnp.float32)
          + b_ref[...])
    if act:
        hn = jnp.where(hn >= 0.0, hn, 0.2 * hn)
    if head is None:
        o_ref[...] = hn
    else:
        cw, cb = head
        pooled = jnp.mean(hn.reshape(_B, _A, _D), axis=1)
        o_ref[...] = (jnp.dot(pooled.astype(jnp.bfloat16),
                              cw[...].astype(jnp.bfloat16),
                              preferred_element_type=jnp.float32) + cb[...])


def _sage_head_body(h_ref, p_ref, wl_ref, wr_ref, b_ref, cw_ref, cb_ref, o_ref):
    _sage_body(h_ref, p_ref, wl_ref, wr_ref, b_ref, o_ref,
               act=False, head=(cw_ref, cb_ref))


def kernel(x, conv1_w, conv1_b, conv2_w, conv2_b,
           sage1_wl, sage1_wr, sage1_b,
           sage2_wl, sage2_wr, sage2_b,
           sage3_wl, sage3_wr, sage3_b,
           cls_w, cls_b):
    xi = x.reshape(_N, _H, _W)
    xp = jnp.pad(xi, ((0, 0), (1, 3), (1, 3)))
    x4 = xp.reshape(_N, 17, 4, 68)
    kh_i = jnp.arange(7)[:, None, None]
    kw_i = jnp.arange(7)[None, :, None]
    ox_i = jnp.arange(16)[None, None, :]
    w1v = jnp.broadcast_to(
        conv1_w.reshape(64, 7, 7).transpose(1, 2, 0)[:, :, None, :],
        (7, 7, 16, 64))
    g = jnp.zeros((7, 68, 16, 64), jnp.float32)
    g = g.at[jnp.broadcast_to(kh_i, (7, 7, 16)),
             jnp.broadcast_to(4 * ox_i + kw_i, (7, 7, 16)),
             jnp.broadcast_to(ox_i, (7, 7, 16)), :].set(w1v)
    g = g.reshape(7, 68, 1024).astype(jnp.bfloat16)
    b1 = jnp.tile(conv1_b, 16).reshape(1, 1024)

    y1 = pl.pallas_call(
        _conv1_body,
        grid=(_N // _C1_BLK,),
        in_specs=[
            pl.BlockSpec((_C1_BLK, 17, 4, 68), lambda i: (i, 0, 0, 0)),
            pl.BlockSpec((7, 68, 1024), lambda i: (0, 0, 0)),
            pl.BlockSpec((1, 1024), lambda i: (0, 0)),
        ],
        out_specs=pl.BlockSpec((_C1_BLK * 16, 1024), lambda i: (i, 0)),
        out_shape=jax.ShapeDtypeStruct((_N * 16, 1024), jnp.float32),
    )(x4, g, b1)

    x6 = y1.reshape(_N, 4, 4, 4, 4, 64)
    w2 = conv2_w.transpose(2, 3, 1, 0).reshape(9, 64, 512)
    b2 = conv2_b.reshape(1, 512)

    enc = pl.pallas_call(
        _conv2_body,
        grid=(_N // _C2_BLK,),
        in_specs=[
            pl.BlockSpec((_C2_BLK, 4, 4, 4, 4, 64),
                         lambda i: (i, 0, 0, 0, 0, 0)),
            pl.BlockSpec((9, 64, 512), lambda i: (0, 0, 0)),
            pl.BlockSpec((1, 512), lambda i: (0, 0)),
        ],
        out_specs=pl.BlockSpec((_C2_BLK, 512), lambda i: (i, 0)),
        out_shape=jax.ShapeDtypeStruct((_N, _D), jnp.float32),
    )(x6, w2, b2)

    p = pl.pallas_call(
        _adj_body,
        out_shape=jax.ShapeDtypeStruct((_B, _A, _A), jnp.float32),
    )(enc.reshape(_B, _A, _D))

    import functools as _ft
    h = enc
    for wl, wr, b in ((sage1_wl, sage1_wr, sage1_b),
                      (sage2_wl, sage2_wr, sage2_b)):
        h = pl.pallas_call(
            _ft.partial(_sage_body, act=True, head=None),
            out_shape=jax.ShapeDtypeStruct((_N, _D), jnp.float32),
        )(h, p, wl.T, wr.T, b.reshape(1, _D))

    out = pl.pallas_call(
        _sage_head_body,
        out_shape=jax.ShapeDtypeStruct((_B, 2), jnp.float32),
    )(h, p, sage3_wl.T, sage3_wr.T, sage3_b.reshape(1, _D),
      cls_w.T, cls_b.reshape(1, 2))
    return out


# speedup vs baseline: 2.8899x; 1.0536x over previous
import jax
import jax.numpy as jnp
from jax.experimental import pallas as pl

_B, _A, _H, _W = 64, 8, 64, 64
_N = _B * _A
_D = 512
_K = 3

_C1_BLK = 16
_C2_BLK = 16


def _conv1_body(x4_ref, g_ref, b_ref, o_ref):
    x4 = x4_ref[...]
    acc = jnp.zeros((_C1_BLK * 16, 1024), jnp.float32)
    for kh in range(7):
        ah, rh = divmod(kh, 4)
        xk = x4[:, ah:ah + 16, rh, :].reshape(_C1_BLK * 16, 68)
        acc = acc + jnp.dot(xk.astype(jnp.bfloat16), g_ref[kh],
                            preferred_element_type=jnp.float32)
    o_ref[...] = jnp.maximum(acc + b_ref[...], 0.0).astype(jnp.bfloat16)


def _conv2_body(x6_ref, w_ref, b_ref, o_ref):
    x6 = x6_ref[...]
    acc = jnp.zeros((_C2_BLK * 16, 512), jnp.float32)
    for kh in range(3):
        for kw in range(3):
            tap = x6[:, :, kh, :, kw, :].reshape(_C2_BLK * 16, 64)
            acc = acc + jnp.dot(tap, w_ref[kh * 3 + kw].astype(jnp.bfloat16),
                                preferred_element_type=jnp.float32)
    acc = jnp.maximum(acc + b_ref[...], 0.0)
    pooled = jnp.mean(acc.reshape(_C2_BLK, 16, 512), axis=1)
    o_ref[...] = pooled


def _adj_body(enc_ref, p_ref):
    jio = jax.lax.broadcasted_iota(jnp.int32, (_B, _A), 1)
    def unit_row(j):
        rj = enc_ref[:, j:j + 1, :]
        ssj = jnp.sum(rj * rj, axis=-1)
        nj = rj / (jnp.sqrt(ssj)[:, :, None] + 1e-12)
        return nj.astype(jnp.bfloat16).astype(jnp.float32)

    units = [unit_row(j) for j in range(_A)]
    for i in range(_A):
        ni = units[i]
        cols = []
        for j in range(_A):
            cols.append(jnp.sum(ni * units[j], axis=-1))
        sims_i = jnp.concatenate(cols, axis=1)
        rank_i = jnp.zeros((_B, _A), jnp.float32)
        for k in range(_A):
            s_k = sims_i[:, k:k + 1]
            gt = (s_k > sims_i).astype(jnp.float32)
            tie = ((s_k == sims_i) & (jio > k)).astype(jnp.float32)
            rank_i = rank_i + gt + tie
        p_i = jnp.where(rank_i < float(_K), 1.0 / _K, 0.0)
        p_ref[:, i:i + 1, :] = p_i[:, None, :]


def _sage_body(h_ref, p_ref, wl_ref, wr_ref, b_ref, o_ref, *, act, head):
    h = h_ref[...]
    h3 = h.reshape(_B, _A, _D)
    rows = []
    for i in range(_A):
        m_i = jnp.zeros((_B, 1, _D), jnp.float32)
        for j in range(_A):
            pij = p_ref[:, i:i + 1, j:j + 1]
            m_i = m_i + pij * h3[:, j:j + 1, :]
        rows.append(m_i)
    m3 = jnp.concatenate(rows, axis=1)
    mean = m3.reshape(_N, _D)
    hn = (jnp.dot(mean.astype(jnp.bfloat16),
                  wl_ref[...].astype(jnp.bfloat16),
                  preferred_element_type=jnp.float32)
          + jnp.dot(h.astype(jnp.bfloat16),
                    wr_ref[...].astype(jnp.bfloat16),
                    preferred_element_type=jnp.float32)
          + b_ref[...])
    if act:
        hn = jnp.where(hn >= 0.0, hn, 0.2 * hn)
    if head is None:
        o_ref[...] = hn
    else:
        cw, cb = head
        pooled = jnp.mean(hn.reshape(_B, _A, _D), axis=1)
        o_ref[...] = (jnp.dot(pooled.astype(jnp.bfloat16),
                              cw[...].astype(jnp.bfloat16),
                              preferred_element_type=jnp.float32) + cb[...])


def _sage_head_body(h_ref, p_ref, wl_ref, wr_ref, b_ref, cw_ref, cb_ref, o_ref):
    _sage_body(h_ref, p_ref, wl_ref, wr_ref, b_ref, o_ref,
               act=False, head=(cw_ref, cb_ref))


def kernel(x, conv1_w, conv1_b, conv2_w, conv2_b,
           sage1_wl, sage1_wr, sage1_b,
           sage2_wl, sage2_wr, sage2_b,
           sage3_wl, sage3_wr, sage3_b,
           cls_w, cls_b):
    xi = x.reshape(_N, _H, _W)
    xp = jnp.pad(xi, ((0, 0), (1, 3), (1, 3)))
    x4 = xp.reshape(_N, 17, 4, 68)
    kh_i = jnp.arange(7)[:, None, None]
    kw_i = jnp.arange(7)[None, :, None]
    ox_i = jnp.arange(16)[None, None, :]
    w1v = jnp.broadcast_to(
        conv1_w.reshape(64, 7, 7).transpose(1, 2, 0)[:, :, None, :],
        (7, 7, 16, 64))
    g = jnp.zeros((7, 68, 16, 64), jnp.float32)
    g = g.at[jnp.broadcast_to(kh_i, (7, 7, 16)),
             jnp.broadcast_to(4 * ox_i + kw_i, (7, 7, 16)),
             jnp.broadcast_to(ox_i, (7, 7, 16)), :].set(w1v)
    g = g.reshape(7, 68, 1024).astype(jnp.bfloat16)
    b1 = jnp.tile(conv1_b, 16).reshape(1, 1024)

    y1 = pl.pallas_call(
        _conv1_body,
        grid=(_N // _C1_BLK,),
        in_specs=[
            pl.BlockSpec((_C1_BLK, 17, 4, 68), lambda i: (i, 0, 0, 0)),
            pl.BlockSpec((7, 68, 1024), lambda i: (0, 0, 0)),
            pl.BlockSpec((1, 1024), lambda i: (0, 0)),
        ],
        out_specs=pl.BlockSpec((_C1_BLK * 16, 1024), lambda i: (i, 0)),
        out_shape=jax.ShapeDtypeStruct((_N * 16, 1024), jnp.bfloat16),
    )(x4, g, b1)

    x6 = y1.reshape(_N, 4, 4, 4, 4, 64)
    w2 = conv2_w.transpose(2, 3, 1, 0).reshape(9, 64, 512)
    b2 = conv2_b.reshape(1, 512)

    enc = pl.pallas_call(
        _conv2_body,
        grid=(_N // _C2_BLK,),
        in_specs=[
            pl.BlockSpec((_C2_BLK, 4, 4, 4, 4, 64),
                         lambda i: (i, 0, 0, 0, 0, 0)),
            pl.BlockSpec((9, 64, 512), lambda i: (0, 0, 0)),
            pl.BlockSpec((1, 512), lambda i: (0, 0)),
        ],
        out_specs=pl.BlockSpec((_C2_BLK, 512), lambda i: (i, 0)),
        out_shape=jax.ShapeDtypeStruct((_N, _D), jnp.float32),
    )(x6, w2, b2)

    p = pl.pallas_call(
        _adj_body,
        out_shape=jax.ShapeDtypeStruct((_B, _A, _A), jnp.float32),
    )(enc.reshape(_B, _A, _D))

    import functools as _ft
    h = enc
    for wl, wr, b in ((sage1_wl, sage1_wr, sage1_b),
                      (sage2_wl, sage2_wr, sage2_b)):
        h = pl.pallas_call(
            _ft.partial(_sage_body, act=True, head=None),
            out_shape=jax.ShapeDtypeStruct((_N, _D), jnp.float32),
        )(h, p, wl.T, wr.T, b.reshape(1, _D))

    out = pl.pallas_call(
        _sage_head_body,
        out_shape=jax.ShapeDtypeStruct((_B, 2), jnp.float32),
    )(h, p, sage3_wl.T, sage3_wr.T, sage3_b.reshape(1, _D),
      cls_w.T, cls_b.reshape(1, 2))
    return out
